# baseline (speedup 1.0000x reference)
import functools

import jax
import jax.numpy as jnp
import numpy as np
from jax.experimental import pallas as pl
from jax.experimental.pallas import tpu as pltpu


_TOPK_RATIO = 0.1


def _attn_block_kernel(q_ref, k_ref, v_ref, o_ref, s_scr, chop_scr, dig_scr,
                       *, k_sel, scale):
    int_min = jnp.int32(-2147483648)
    q = q_ref[0]
    k = k_ref[0]
    v = v_ref[0]
    st = jax.lax.dot_general(k, q, (((1,), (1,)), ((), ())),
                             preferred_element_type=jnp.float32) * scale
    s_scr[...] = st
    ikeys = jax.lax.bitcast_convert_type(st, jnp.int32)
    mono = ikeys ^ (jax.lax.shift_right_arithmetic(ikeys, 31)
                    & jnp.int32(0x7FFFFFFF))
    chop_scr[...] = jax.lax.bitcast_convert_type(
        jax.lax.shift_right_logical(ikeys, 16).astype(jnp.int16),
        jnp.bfloat16)
    dig_scr[...] = (jax.lax.shift_right_logical(mono, 8)
                    & jnp.int32(0xFF)).astype(jnp.bfloat16)

    ss, bq = st.shape
    kf = jnp.float32(k_sel)
    one_bf = jnp.bfloat16(1.0)
    zero_bf = jnp.bfloat16(0.0)
    chunk = 64
    exact_bf = ss <= 2048

    def _count_ge(arr_ref, cand_bf):
        acc2 = None
        for j in range(ss // chunk):
            onz = jnp.where(arr_ref[j * chunk:(j + 1) * chunk, :] >= cand_bf,
                            one_bf, zero_bf)
            sub = onz[:16]
            for i in range(1, chunk // 16):
                sub = sub + onz[i * 16:(i + 1) * 16]
            if not exact_bf:
                sub = sub.astype(jnp.float32)
            acc2 = sub if acc2 is None else acc2 + sub
        return jnp.sum(acc2.astype(jnp.float32), axis=0, keepdims=True)

    def _u16_to_bf(u):
        m = u ^ jnp.int32(0x8000)
        flip = (jax.lax.shift_right_logical(m, 15) & jnp.int32(1)) * \
            jnp.int32(0x7FFF)
        return jax.lax.bitcast_convert_type(
            (m ^ flip).astype(jnp.int16), jnp.bfloat16)

    def body_hi(i, t):
        bit = jnp.left_shift(jnp.int32(1), 15 - i)
        cand_u = t | bit
        cnt = _count_ge(chop_scr, _u16_to_bf(cand_u))
        return jnp.where(cnt >= kf, cand_u, t)

    t0 = jnp.zeros((1, bq), jnp.int32)
    t_hi = jax.lax.fori_loop(0, 16, body_hi, t0, unroll=16)

    t_chop = _u16_to_bf(t_hi)
    gacc = None
    for j in range(ss // chunk):
        ong = jnp.where(chop_scr[j * chunk:(j + 1) * chunk, :] > t_chop,
                        one_bf, zero_bf)
        sub = ong[:16]
        for i in range(1, chunk // 16):
            sub = sub + ong[i * 16:(i + 1) * 16]
        if not exact_bf:
            sub = sub.astype(jnp.float32)
        gacc = sub if gacc is None else gacc + sub
    gt_cnt = jnp.sum(gacc.astype(jnp.float32), axis=0, keepdims=True)
    dig_scr[...] = jnp.where(chop_scr[...] == t_chop, dig_scr[...],
                             jnp.bfloat16(-1.0))

    def body_lo(i, d):
        bit = jnp.left_shift(jnp.int32(1), 7 - i)
        cand_d = d | bit
        cnt = gt_cnt + _count_ge(dig_scr, cand_d.astype(jnp.bfloat16))
        return jnp.where(cnt >= kf, cand_d, d)

    d = jax.lax.fori_loop(0, 8, body_lo, t0, unroll=8)

    t = jnp.left_shift(t_hi, 16) | jnp.left_shift(d, 8)
    thresh_s = t ^ int_min
    thresh_i = thresh_s ^ (jax.lax.shift_right_arithmetic(thresh_s, 31)
                           & jnp.int32(0x7FFFFFFF))
    thresh_f = jax.lax.bitcast_convert_type(thresh_i, jnp.float32)

    st = s_scr[...]
    neg = jnp.finfo(jnp.float32).min
    masked = jnp.where(st >= thresh_f, st, neg)
    m = jnp.max(masked, axis=0, keepdims=True)
    e = jnp.exp(masked - m)
    denom = jnp.sum(e, axis=0, keepdims=True)
    p = (e / denom).astype(jnp.bfloat16)
    o = jax.lax.dot_general(p, v.astype(jnp.bfloat16),
                            (((0,), (0,)), ((), ())),
                            preferred_element_type=jnp.float32)
    o_ref[0] = o


def kernel(query, key, value):
    B, S, H, D = query.shape
    assert B == 1
    k_sel = max(1, int(_TOPK_RATIO * S))
    scale = 1.0 / float(np.sqrt(D))
    BQ = 2048
    while S % BQ:
        BQ //= 2
    NQ = S // BQ

    q3 = query[0].transpose(1, 0, 2)
    k3 = key[0].transpose(1, 0, 2)
    v3 = value[0].transpose(1, 0, 2)

    grid = (H, NQ)
    out = pl.pallas_call(
        functools.partial(_attn_block_kernel, k_sel=k_sel, scale=scale),
        grid=grid,
        in_specs=[
            pl.BlockSpec((1, BQ, D), lambda h, qb: (h, qb, 0)),
            pl.BlockSpec((1, S, D), lambda h, qb: (h, 0, 0)),
            pl.BlockSpec((1, S, D), lambda h, qb: (h, 0, 0)),
        ],
        out_specs=pl.BlockSpec((1, BQ, D), lambda h, qb: (0, qb, h)),
        out_shape=jax.ShapeDtypeStruct((1, S, H * D), jnp.float32),
        scratch_shapes=[
            pltpu.VMEM((S, BQ), jnp.float32),
            pltpu.VMEM((S, BQ), jnp.bfloat16),
            pltpu.VMEM((S, BQ), jnp.bfloat16),
        ],
    )(q3, k3, v3)
    return out


# speedup vs baseline: 1.2545x; 1.2545x over previous
import functools

import jax
import jax.numpy as jnp
import numpy as np
from jax.experimental import pallas as pl
from jax.experimental.pallas import tpu as pltpu


_TOPK_RATIO = 0.1


def _attn_block_kernel(q_ref, k_ref, v_ref, o_ref, s_scr, chop_scr, dig_scr,
                       *, k_sel, scale):
    int_min = jnp.int32(-2147483648)
    q = q_ref[0]
    k = k_ref[0]
    v = v_ref[0]
    st = jax.lax.dot_general(k, q, (((1,), (1,)), ((), ())),
                             preferred_element_type=jnp.float32) * scale
    s_scr[...] = st
    ikeys = jax.lax.bitcast_convert_type(st, jnp.int32)
    mono = ikeys ^ (jax.lax.shift_right_arithmetic(ikeys, 31)
                    & jnp.int32(0x7FFFFFFF))
    chop_scr[...] = jax.lax.bitcast_convert_type(
        jax.lax.shift_right_logical(ikeys, 16).astype(jnp.int16),
        jnp.bfloat16)
    dig_scr[...] = (jax.lax.shift_right_logical(mono, 8)
                    & jnp.int32(0xFF)).astype(jnp.bfloat16)

    ss, bq = st.shape
    kf = jnp.float32(k_sel)
    one_bf = jnp.bfloat16(1.0)
    zero_bf = jnp.bfloat16(0.0)
    chunk = 128
    exact_bf = ss <= 2048

    def _count_ge(arr_ref, cand_bf):
        acc2 = None
        for j in range(ss // chunk):
            onz = jnp.where(arr_ref[j * chunk:(j + 1) * chunk, :] >= cand_bf,
                            one_bf, zero_bf)
            sub = onz[:16]
            for i in range(1, chunk // 16):
                sub = sub + onz[i * 16:(i + 1) * 16]
            if not exact_bf:
                sub = sub.astype(jnp.float32)
            acc2 = sub if acc2 is None else acc2 + sub
        return jnp.sum(acc2.astype(jnp.float32), axis=0, keepdims=True)

    def _u16_to_bf(u):
        m = u ^ jnp.int32(0x8000)
        flip = (jax.lax.shift_right_logical(m, 15) & jnp.int32(1)) * \
            jnp.int32(0x7FFF)
        return jax.lax.bitcast_convert_type(
            (m ^ flip).astype(jnp.int16), jnp.bfloat16)

    def body_hi(i, t):
        bit = jnp.left_shift(jnp.int32(1), 15 - i)
        cand_u = t | bit
        cnt = _count_ge(chop_scr, _u16_to_bf(cand_u))
        return jnp.where(cnt >= kf, cand_u, t)

    t0 = jnp.zeros((1, bq), jnp.int32)
    t_hi = jax.lax.fori_loop(0, 16, body_hi, t0, unroll=16)

    t_chop = _u16_to_bf(t_hi)
    gacc = None
    for j in range(ss // chunk):
        ong = jnp.where(chop_scr[j * chunk:(j + 1) * chunk, :] > t_chop,
                        one_bf, zero_bf)
        sub = ong[:16]
        for i in range(1, chunk // 16):
            sub = sub + ong[i * 16:(i + 1) * 16]
        if not exact_bf:
            sub = sub.astype(jnp.float32)
        gacc = sub if gacc is None else gacc + sub
    gt_cnt = jnp.sum(gacc.astype(jnp.float32), axis=0, keepdims=True)
    dig_scr[...] = jnp.where(chop_scr[...] == t_chop, dig_scr[...],
                             jnp.bfloat16(-1.0))

    def body_lo(i, d):
        bit = jnp.left_shift(jnp.int32(1), 7 - i)
        cand_d = d | bit
        cnt = gt_cnt + _count_ge(dig_scr, cand_d.astype(jnp.bfloat16))
        return jnp.where(cnt >= kf, cand_d, d)

    d = jax.lax.fori_loop(0, 8, body_lo, t0, unroll=8)

    t = jnp.left_shift(t_hi, 16) | jnp.left_shift(d, 8)
    thresh_s = t ^ int_min
    thresh_i = thresh_s ^ (jax.lax.shift_right_arithmetic(thresh_s, 31)
                           & jnp.int32(0x7FFFFFFF))
    thresh_f = jax.lax.bitcast_convert_type(thresh_i, jnp.float32)

    st = s_scr[...]
    neg = jnp.finfo(jnp.float32).min
    masked = jnp.where(st >= thresh_f, st, neg)
    m = jnp.max(masked, axis=0, keepdims=True)
    e = jnp.exp(masked - m)
    denom = jnp.sum(e, axis=0, keepdims=True)
    p = (e / denom).astype(jnp.bfloat16)
    o = jax.lax.dot_general(p, v.astype(jnp.bfloat16),
                            (((0,), (0,)), ((), ())),
                            preferred_element_type=jnp.float32)
    o_ref[0] = o


def kernel(query, key, value):
    B, S, H, D = query.shape
    assert B == 1
    k_sel = max(1, int(_TOPK_RATIO * S))
    scale = 1.0 / float(np.sqrt(D))
    BQ = 1024
    while S % BQ:
        BQ //= 2
    NQ = S // BQ

    q3 = query[0].transpose(1, 0, 2)
    k3 = key[0].transpose(1, 0, 2)
    v3 = value[0].transpose(1, 0, 2)

    grid = (H, NQ)
    out = pl.pallas_call(
        functools.partial(_attn_block_kernel, k_sel=k_sel, scale=scale),
        grid=grid,
        in_specs=[
            pl.BlockSpec((1, BQ, D), lambda h, qb: (h, qb, 0)),
            pl.BlockSpec((1, S, D), lambda h, qb: (h, 0, 0)),
            pl.BlockSpec((1, S, D), lambda h, qb: (h, 0, 0)),
        ],
        out_specs=pl.BlockSpec((1, BQ, D), lambda h, qb: (0, qb, h)),
        out_shape=jax.ShapeDtypeStruct((1, S, H * D), jnp.float32),
        scratch_shapes=[
            pltpu.VMEM((S, BQ), jnp.float32),
            pltpu.VMEM((S, BQ), jnp.bfloat16),
            pltpu.VMEM((S, BQ), jnp.bfloat16),
        ],
    )(q3, k3, v3)
    return out


# speedup vs baseline: 1.2628x; 1.0066x over previous
import functools

import jax
import jax.numpy as jnp
import numpy as np
from jax.experimental import pallas as pl
from jax.experimental.pallas import tpu as pltpu


_TOPK_RATIO = 0.1


def _attn_block_kernel(q_ref, k_ref, v_ref, o_ref, s_scr, chop_scr, dig_scr,
                       *, k_sel, scale):
    int_min = jnp.int32(-2147483648)
    q = q_ref[0]
    k = k_ref[0]
    v = v_ref[0]
    st = jax.lax.dot_general(k, q, (((1,), (1,)), ((), ())),
                             preferred_element_type=jnp.float32) * scale
    s_scr[...] = st
    ikeys = jax.lax.bitcast_convert_type(st, jnp.int32)
    mono = ikeys ^ (jax.lax.shift_right_arithmetic(ikeys, 31)
                    & jnp.int32(0x7FFFFFFF))
    chop_scr[...] = jax.lax.bitcast_convert_type(
        jax.lax.shift_right_logical(ikeys, 16).astype(jnp.int16),
        jnp.bfloat16)
    dig_scr[...] = (jax.lax.shift_right_logical(mono, 8)
                    & jnp.int32(0xFF)).astype(jnp.bfloat16)

    ss, bq = st.shape
    kf = jnp.float32(k_sel)
    one_bf = jnp.bfloat16(1.0)
    zero_bf = jnp.bfloat16(0.0)
    chunk = 64
    exact_bf = ss <= 2048

    def _count_ge(arr_ref, cand_bf):
        acc2 = None
        for j in range(ss // chunk):
            onz = jnp.where(arr_ref[j * chunk:(j + 1) * chunk, :] >= cand_bf,
                            one_bf, zero_bf)
            sub = onz[:8]
            for i in range(1, chunk // 8):
                sub = sub + onz[i * 8:(i + 1) * 8]
            if not exact_bf:
                sub = sub.astype(jnp.float32)
            acc2 = sub if acc2 is None else acc2 + sub
        return jnp.sum(acc2.astype(jnp.float32), axis=0, keepdims=True)

    def _u16_to_bf(u):
        m = u ^ jnp.int32(0x8000)
        flip = (jax.lax.shift_right_logical(m, 15) & jnp.int32(1)) * \
            jnp.int32(0x7FFF)
        return jax.lax.bitcast_convert_type(
            (m ^ flip).astype(jnp.int16), jnp.bfloat16)

    def body_hi(i, t):
        bit = jnp.left_shift(jnp.int32(1), 15 - i)
        cand_u = t | bit
        cnt = _count_ge(chop_scr, _u16_to_bf(cand_u))
        return jnp.where(cnt >= kf, cand_u, t)

    t0 = jnp.zeros((1, bq), jnp.int32)
    t_hi = jax.lax.fori_loop(0, 16, body_hi, t0, unroll=16)

    t_chop = _u16_to_bf(t_hi)
    gacc = None
    for j in range(ss // chunk):
        ong = jnp.where(chop_scr[j * chunk:(j + 1) * chunk, :] > t_chop,
                        one_bf, zero_bf)
        sub = ong[:8]
        for i in range(1, chunk // 8):
            sub = sub + ong[i * 8:(i + 1) * 8]
        if not exact_bf:
            sub = sub.astype(jnp.float32)
        gacc = sub if gacc is None else gacc + sub
    gt_cnt = jnp.sum(gacc.astype(jnp.float32), axis=0, keepdims=True)
    dig_scr[...] = jnp.where(chop_scr[...] == t_chop, dig_scr[...],
                             jnp.bfloat16(-1.0))

    def body_lo(i, d):
        bit = jnp.left_shift(jnp.int32(1), 7 - i)
        cand_d = d | bit
        cnt = gt_cnt + _count_ge(dig_scr, cand_d.astype(jnp.bfloat16))
        return jnp.where(cnt >= kf, cand_d, d)

    d = jax.lax.fori_loop(0, 8, body_lo, t0, unroll=8)

    t = jnp.left_shift(t_hi, 16) | jnp.left_shift(d, 8)
    thresh_s = t ^ int_min
    thresh_i = thresh_s ^ (jax.lax.shift_right_arithmetic(thresh_s, 31)
                           & jnp.int32(0x7FFFFFFF))
    thresh_f = jax.lax.bitcast_convert_type(thresh_i, jnp.float32)

    st = s_scr[...]
    neg = jnp.finfo(jnp.float32).min
    masked = jnp.where(st >= thresh_f, st, neg)
    m = jnp.max(masked, axis=0, keepdims=True)
    e = jnp.exp(masked - m)
    denom = jnp.sum(e, axis=0, keepdims=True)
    p = (e / denom).astype(jnp.bfloat16)
    o = jax.lax.dot_general(p, v.astype(jnp.bfloat16),
                            (((0,), (0,)), ((), ())),
                            preferred_element_type=jnp.float32)
    o_ref[0] = o


def kernel(query, key, value):
    B, S, H, D = query.shape
    assert B == 1
    k_sel = max(1, int(_TOPK_RATIO * S))
    scale = 1.0 / float(np.sqrt(D))
    BQ = 1024
    while S % BQ:
        BQ //= 2
    NQ = S // BQ

    q3 = query[0].transpose(1, 0, 2)
    k3 = key[0].transpose(1, 0, 2)
    v3 = value[0].transpose(1, 0, 2)

    grid = (H, NQ)
    out = pl.pallas_call(
        functools.partial(_attn_block_kernel, k_sel=k_sel, scale=scale),
        grid=grid,
        in_specs=[
            pl.BlockSpec((1, BQ, D), lambda h, qb: (h, qb, 0)),
            pl.BlockSpec((1, S, D), lambda h, qb: (h, 0, 0)),
            pl.BlockSpec((1, S, D), lambda h, qb: (h, 0, 0)),
        ],
        out_specs=pl.BlockSpec((1, BQ, D), lambda h, qb: (0, qb, h)),
        out_shape=jax.ShapeDtypeStruct((1, S, H * D), jnp.float32),
        scratch_shapes=[
            pltpu.VMEM((S, BQ), jnp.float32),
            pltpu.VMEM((S, BQ), jnp.bfloat16),
            pltpu.VMEM((S, BQ), jnp.bfloat16),
        ],
    )(q3, k3, v3)
    return out


# speedup vs baseline: 1.2795x; 1.0133x over previous
import functools

import jax
import jax.numpy as jnp
import numpy as np
from jax.experimental import pallas as pl
from jax.experimental.pallas import tpu as pltpu


_TOPK_RATIO = 0.1


def _attn_block_kernel(q_ref, k_ref, v_ref, o_ref, s_scr, chop_scr, dig_scr,
                       *, k_sel, scale):
    int_min = jnp.int32(-2147483648)
    q = q_ref[0]
    k = k_ref[0]
    v = v_ref[0]
    st = jax.lax.dot_general(k, q, (((1,), (1,)), ((), ())),
                             preferred_element_type=jnp.float32) * scale
    s_scr[...] = st
    ikeys = jax.lax.bitcast_convert_type(st, jnp.int32)
    mono = ikeys ^ (jax.lax.shift_right_arithmetic(ikeys, 31)
                    & jnp.int32(0x7FFFFFFF))
    chop_scr[...] = jax.lax.bitcast_convert_type(
        jax.lax.shift_right_logical(ikeys, 16).astype(jnp.int16),
        jnp.bfloat16)
    dig_scr[...] = (jax.lax.shift_right_logical(mono, 8)
                    & jnp.int32(0xFF)).astype(jnp.bfloat16)

    ss, bq = st.shape
    kf = jnp.float32(k_sel)
    one_bf = jnp.bfloat16(1.0)
    zero_bf = jnp.bfloat16(0.0)
    chunk = 64
    exact_bf = ss <= 2048

    def _count_ge(arr_ref, cand_bf):
        acc2 = None
        for j in range(ss // chunk):
            onz = jnp.where(arr_ref[j * chunk:(j + 1) * chunk, :] >= cand_bf,
                            one_bf, zero_bf)
            sub = onz[:8]
            for i in range(1, chunk // 8):
                sub = sub + onz[i * 8:(i + 1) * 8]
            if not exact_bf:
                sub = sub.astype(jnp.float32)
            acc2 = sub if acc2 is None else acc2 + sub
        return jnp.sum(acc2.astype(jnp.float32), axis=0, keepdims=True)

    def _u16_to_bf(u):
        m = u ^ jnp.int32(0x8000)
        flip = (jax.lax.shift_right_logical(m, 15) & jnp.int32(1)) * \
            jnp.int32(0x7FFF)
        return jax.lax.bitcast_convert_type(
            (m ^ flip).astype(jnp.int16), jnp.bfloat16)

    def body_hi(i, t):
        bit = jnp.left_shift(jnp.int32(1), 15 - i)
        cand_u = t | bit
        cnt = _count_ge(chop_scr, _u16_to_bf(cand_u))
        return jnp.where(cnt >= kf, cand_u, t)

    t0 = jnp.zeros((1, bq), jnp.int32)
    t_hi = jax.lax.fori_loop(0, 16, body_hi, t0, unroll=16)

    t_chop = _u16_to_bf(t_hi)
    gacc = None
    for j in range(ss // chunk):
        cchunk = chop_scr[j * chunk:(j + 1) * chunk, :]
        ong = jnp.where(cchunk > t_chop, one_bf, zero_bf)
        dig_scr[j * chunk:(j + 1) * chunk, :] = jnp.where(
            cchunk == t_chop, dig_scr[j * chunk:(j + 1) * chunk, :],
            jnp.bfloat16(-1.0))
        sub = ong[:8]
        for i in range(1, chunk // 8):
            sub = sub + ong[i * 8:(i + 1) * 8]
        if not exact_bf:
            sub = sub.astype(jnp.float32)
        gacc = sub if gacc is None else gacc + sub
    gt_cnt = jnp.sum(gacc.astype(jnp.float32), axis=0, keepdims=True)

    def body_lo(i, d):
        bit = jnp.left_shift(jnp.int32(1), 7 - i)
        cand_d = d | bit
        cnt = gt_cnt + _count_ge(dig_scr, cand_d.astype(jnp.bfloat16))
        return jnp.where(cnt >= kf, cand_d, d)

    d = jax.lax.fori_loop(0, 8, body_lo, t0, unroll=8)

    t = jnp.left_shift(t_hi, 16) | jnp.left_shift(d, 8)
    thresh_s = t ^ int_min
    thresh_i = thresh_s ^ (jax.lax.shift_right_arithmetic(thresh_s, 31)
                           & jnp.int32(0x7FFFFFFF))
    thresh_f = jax.lax.bitcast_convert_type(thresh_i, jnp.float32)

    st = s_scr[...]
    neg = jnp.finfo(jnp.float32).min
    masked = jnp.where(st >= thresh_f, st, neg)
    m = jnp.max(masked, axis=0, keepdims=True)
    e = jnp.exp(masked - m)
    denom = jnp.sum(e, axis=0, keepdims=True)
    p = (e / denom).astype(jnp.bfloat16)
    o = jax.lax.dot_general(p, v.astype(jnp.bfloat16),
                            (((0,), (0,)), ((), ())),
                            preferred_element_type=jnp.float32)
    o_ref[0] = o


def kernel(query, key, value):
    B, S, H, D = query.shape
    assert B == 1
    k_sel = max(1, int(_TOPK_RATIO * S))
    scale = 1.0 / float(np.sqrt(D))
    BQ = 1024
    while S % BQ:
        BQ //= 2
    NQ = S // BQ

    q3 = query[0].transpose(1, 0, 2)
    k3 = key[0].transpose(1, 0, 2)
    v3 = value[0].transpose(1, 0, 2)

    grid = (H, NQ)
    out = pl.pallas_call(
        functools.partial(_attn_block_kernel, k_sel=k_sel, scale=scale),
        grid=grid,
        in_specs=[
            pl.BlockSpec((1, BQ, D), lambda h, qb: (h, qb, 0)),
            pl.BlockSpec((1, S, D), lambda h, qb: (h, 0, 0)),
            pl.BlockSpec((1, S, D), lambda h, qb: (h, 0, 0)),
        ],
        out_specs=pl.BlockSpec((1, BQ, D), lambda h, qb: (0, qb, h)),
        out_shape=jax.ShapeDtypeStruct((1, S, H * D), jnp.float32),
        scratch_shapes=[
            pltpu.VMEM((S, BQ), jnp.float32),
            pltpu.VMEM((S, BQ), jnp.bfloat16),
            pltpu.VMEM((S, BQ), jnp.bfloat16),
        ],
    )(q3, k3, v3)
    return out
